# Initial kernel scaffold; baseline (speedup 1.0000x reference)
import functools

import jax
import jax.numpy as jnp
from jax.experimental import pallas as pl
from jax.experimental.pallas import tpu as pltpu

_INTERPRET = False

_D = 32
_IMG = 64
_C = 3


def _decode_body(zw_ref, w_ref, b_ref, out_ref):
    x = jnp.dot(zw_ref[...], w_ref[...],
                preferred_element_type=jnp.float32,
                precision=jax.lax.Precision.HIGHEST)
    out_ref[...] = jax.nn.sigmoid(x + b_ref[...][None, :])


def _render_body(glyphs_ref, zwhere_ref, zpres_ref, zdepth_ref, idx_ref,
                 out_ref, bk_ref):
    b = pl.program_id(0)
    nf = zwhere_ref.shape[1]
    out_ref[...] = jnp.zeros(out_ref.shape, jnp.float32)
    bk_ref[...] = jnp.full(bk_ref.shape, -jnp.inf, jnp.float32)

    gx = jax.lax.broadcasted_iota(jnp.float32, (1, _IMG), 1) * (2.0 / (_IMG - 1)) - 1.0
    gy = jax.lax.broadcasted_iota(jnp.float32, (_IMG, 1), 0) * (2.0 / (_IMG - 1)) - 1.0
    xp = jax.lax.broadcasted_iota(jnp.float32, (_D, 1), 0)
    yp = jax.lax.broadcasted_iota(jnp.float32, (1, _D), 1)

    def body(j, carry):
        pres = zpres_ref[b, j, 0]

        @pl.when(pres == 1)
        def _():
            loc = idx_ref[j]
            key = zdepth_ref[b, loc, 0]
            cx = zwhere_ref[b, j, 0]
            cy = zwhere_ref[b, j, 1]
            w = zwhere_ref[b, j, 2]
            h = zwhere_ref[b, j, 3]
            rw = 1.0 / (w + 1e-5)
            rh = 1.0 / (h + 1e-5)
            sx = ((gx - (2.0 * cx - 1.0)) * rw + 1.0) * ((_D - 1) / 2.0)
            sy = ((gy - (2.0 * cy - 1.0)) * rh + 1.0) * ((_D - 1) / 2.0)
            rxt = jnp.maximum(0.0, 1.0 - jnp.abs(sx - xp))
            ry = jnp.maximum(0.0, 1.0 - jnp.abs(sy - yp))
            g = glyphs_ref[0, loc]
            a = jnp.dot(g, rxt, preferred_element_type=jnp.float32,
                        precision=jax.lax.Precision.HIGHEST)
            for c in range(_C):
                r_c = jnp.dot(ry, a[c * _D:(c + 1) * _D, :],
                              preferred_element_type=jnp.float32,
                              precision=jax.lax.Precision.HIGHEST)
                better = (r_c != 0.0) & (key > bk_ref[c])
                out_ref[0, c] = jnp.where(better, r_c, out_ref[0, c])
                bk_ref[c] = jnp.where(better, key, bk_ref[c])

        return carry

    jax.lax.fori_loop(0, nf, body, 0)


def kernel(z_what, z_where, z_present, z_depth, indices, W_dec, b_dec):
    B, NL, Z = z_what.shape
    NF = z_where.shape[1]

    decoded = pl.pallas_call(
        _decode_body,
        out_shape=jax.ShapeDtypeStruct((B * NL, _C * _D * _D), jnp.float32),
        interpret=_INTERPRET,
    )(z_what.reshape(B * NL, Z), W_dec, b_dec)
    glyphs = decoded.reshape(B, NL, _C * _D, _D)

    out = pl.pallas_call(
        _render_body,
        grid=(B,),
        in_specs=[
            pl.BlockSpec((1, NL, _C * _D, _D), lambda b: (b, 0, 0, 0)),
            pl.BlockSpec(memory_space=pltpu.SMEM),
            pl.BlockSpec(memory_space=pltpu.SMEM),
            pl.BlockSpec(memory_space=pltpu.SMEM),
            pl.BlockSpec(memory_space=pltpu.SMEM),
        ],
        out_specs=pl.BlockSpec((1, _C, _IMG, _IMG), lambda b: (b, 0, 0, 0)),
        out_shape=jax.ShapeDtypeStruct((B, _C, _IMG, _IMG), jnp.float32),
        scratch_shapes=[pltpu.VMEM((_C, _IMG, _IMG), jnp.float32)],
        interpret=_INTERPRET,
    )(glyphs, z_where, z_present, z_depth, indices)
    return out


# speedup vs baseline: 1604.2914x; 1604.2914x over previous
import functools

import jax
import jax.numpy as jnp
from jax.experimental import pallas as pl
from jax.experimental.pallas import tpu as pltpu

_INTERPRET = False

_D = 32
_IMG = 64
_C = 3


def _decode_body(zw_ref, w_ref, b_ref, out_ref):
    x = jnp.dot(zw_ref[...], w_ref[...],
                preferred_element_type=jnp.float32,
                precision=jax.lax.Precision.HIGHEST)
    out_ref[...] = jax.nn.sigmoid(x + b_ref[...][None, :])


def _render_body(glyphs_ref, zwhere_ref, zpres_ref, zdepth_ref, idx_ref,
                 out_ref, bk_ref):
    b = pl.program_id(0)
    nf = zwhere_ref.shape[1]
    out_ref[...] = jnp.zeros(out_ref.shape, jnp.float32)
    bk_ref[...] = jnp.full(bk_ref.shape, -jnp.inf, jnp.float32)

    gx = jax.lax.broadcasted_iota(jnp.int32, (1, _IMG), 1).astype(jnp.float32) * (2.0 / (_IMG - 1)) - 1.0
    gy = jax.lax.broadcasted_iota(jnp.int32, (_IMG, 1), 0).astype(jnp.float32) * (2.0 / (_IMG - 1)) - 1.0
    xp = jax.lax.broadcasted_iota(jnp.int32, (_D, 1), 0).astype(jnp.float32)
    yp = jax.lax.broadcasted_iota(jnp.int32, (1, _D), 1).astype(jnp.float32)

    def body(j, carry):
        pres = zpres_ref[b, j, 0]

        @pl.when(pres == 1)
        def _():
            loc = idx_ref[j]
            key = zdepth_ref[b, loc, 0]
            cx = zwhere_ref[b, j, 0]
            cy = zwhere_ref[b, j, 1]
            w = zwhere_ref[b, j, 2]
            h = zwhere_ref[b, j, 3]
            rw = 1.0 / (w + 1e-5)
            rh = 1.0 / (h + 1e-5)
            sx = ((gx - (2.0 * cx - 1.0)) * rw + 1.0) * ((_D - 1) / 2.0)
            sy = ((gy - (2.0 * cy - 1.0)) * rh + 1.0) * ((_D - 1) / 2.0)
            rxt = jnp.maximum(0.0, 1.0 - jnp.abs(sx - xp))
            ry = jnp.maximum(0.0, 1.0 - jnp.abs(sy - yp))
            g = glyphs_ref[0, loc]
            a = jnp.dot(g, rxt, preferred_element_type=jnp.float32,
                        precision=jax.lax.Precision.HIGHEST)
            for c in range(_C):
                r_c = jnp.dot(ry, a[c * _D:(c + 1) * _D, :],
                              preferred_element_type=jnp.float32,
                              precision=jax.lax.Precision.HIGHEST)
                better = (r_c != 0.0) & (key > bk_ref[c])
                out_ref[0, c] = jnp.where(better, r_c, out_ref[0, c])
                bk_ref[c] = jnp.where(better, key, bk_ref[c])

        return carry

    jax.lax.fori_loop(0, nf, body, 0)


def kernel(z_what, z_where, z_present, z_depth, indices, W_dec, b_dec):
    B, NL, Z = z_what.shape
    NF = z_where.shape[1]

    decoded = pl.pallas_call(
        _decode_body,
        out_shape=jax.ShapeDtypeStruct((B * NL, _C * _D * _D), jnp.float32),
        interpret=_INTERPRET,
    )(z_what.reshape(B * NL, Z), W_dec, b_dec)
    glyphs = decoded.reshape(B, NL, _C * _D, _D)

    out = pl.pallas_call(
        _render_body,
        grid=(B,),
        in_specs=[
            pl.BlockSpec((1, NL, _C * _D, _D), lambda b: (b, 0, 0, 0)),
            pl.BlockSpec(memory_space=pltpu.SMEM),
            pl.BlockSpec(memory_space=pltpu.SMEM),
            pl.BlockSpec(memory_space=pltpu.SMEM),
            pl.BlockSpec(memory_space=pltpu.SMEM),
        ],
        out_specs=pl.BlockSpec((1, _C, _IMG, _IMG), lambda b: (b, 0, 0, 0)),
        out_shape=jax.ShapeDtypeStruct((B, _C, _IMG, _IMG), jnp.float32),
        scratch_shapes=[pltpu.VMEM((_C, _IMG, _IMG), jnp.float32)],
        interpret=_INTERPRET,
    )(glyphs, z_where, z_present, z_depth, indices)
    return out
